# Initial kernel scaffold; baseline (speedup 1.0000x reference)
import functools

import jax
import jax.numpy as jnp
from jax import lax
from jax.experimental import pallas as pl
from jax.experimental.pallas import tpu as pltpu
from jax.experimental.pallas import tpu_sc as plsc

NUM_FIELDS = 26
VOCAB = 1000
EMB = 32
BATCH = 4096

_INFO = plsc.get_sparse_core_info()
_NC, _NS, _L = _INFO.num_cores, _INFO.num_subcores, _INFO.num_lanes
_NW = _NC * _NS
_TOTAL = BATCH * NUM_FIELDS
_PER_W = _TOTAL // _NW
_CHUNK = 128
_NCHUNK = _PER_W // _CHUNK


def _make_kernel():
  mesh = plsc.VectorSubcoreMesh(core_axis_name="c", subcore_axis_name="s")

  @functools.partial(
      pl.kernel,
      out_type=jax.ShapeDtypeStruct((_TOTAL, EMB), jnp.float32),
      mesh=mesh,
      scratch_types=[
          pltpu.VMEM((_PER_W,), jnp.int32),
          pltpu.VMEM((_PER_W, EMB), jnp.float32),
          pltpu.SemaphoreType.DMA,
      ],
  )
  def body(tables_hbm, sf_hbm, out_hbm, idx_v, rows_v, sem):
    wid = lax.axis_index("s") * _NC + lax.axis_index("c")
    base = wid * _PER_W

    pltpu.sync_copy(sf_hbm.at[pl.ds(base, _PER_W)], idx_v)

    def add_offsets(i, carry):
      lanes = base + i * _L + lax.iota(jnp.int32, _L)
      f = lax.rem(lanes, NUM_FIELDS)
      sl = pl.ds(i * _L, _L)
      idx_v[sl] = idx_v[sl] + f * VOCAB
      return carry

    lax.fori_loop(0, _PER_W // _L, add_offsets, 0)

    copies = []
    for j in range(_NCHUNK):
      sl = pl.ds(j * _CHUNK, _CHUNK)
      copies.append(
          pltpu.async_copy(tables_hbm.at[idx_v.at[sl]], rows_v.at[sl], sem)
      )
    for c in copies:
      c.wait()

    pltpu.sync_copy(rows_v, out_hbm.at[pl.ds(base, _PER_W)])

  return body


_gather_kernel = _make_kernel()


@jax.jit
def kernel(sparse_features, tables):
  tables_flat = tables.reshape(NUM_FIELDS * VOCAB, EMB)
  sf_flat = sparse_features.reshape(_TOTAL)
  out_flat = _gather_kernel(tables_flat, sf_flat)
  return out_flat.reshape(BATCH, NUM_FIELDS * EMB)


# speedup vs baseline: 6.9041x; 6.9041x over previous
import functools

import jax
import jax.numpy as jnp
from jax import lax
from jax.experimental import pallas as pl
from jax.experimental.pallas import tpu as pltpu
from jax.experimental.pallas import tpu_sc as plsc

NUM_FIELDS = 26
VOCAB = 1000
EMB = 32
BATCH = 4096

_INFO = plsc.get_sparse_core_info()
_NC, _NS, _L = _INFO.num_cores, _INFO.num_subcores, _INFO.num_lanes
_NW = _NC * _NS
_TOTAL = BATCH * NUM_FIELDS
_PER_W = _TOTAL // _NW
_CHUNK = 128
_NCHUNK = _PER_W // _CHUNK


def _make_kernel():
  mesh = plsc.VectorSubcoreMesh(core_axis_name="c", subcore_axis_name="s")

  @functools.partial(
      pl.kernel,
      out_type=jax.ShapeDtypeStruct((_TOTAL, EMB), jnp.float32),
      mesh=mesh,
      scratch_types=[
          pltpu.VMEM((_PER_W,), jnp.int32),
          pltpu.VMEM((_PER_W, EMB), jnp.float32),
          pltpu.SemaphoreType.DMA,
      ],
      compiler_params=pltpu.CompilerParams(use_tc_tiling_on_sc=False),
  )
  def body(tables_hbm, sf_hbm, out_hbm, idx_v, rows_v, sem):
    wid = lax.axis_index("s") * _NC + lax.axis_index("c")
    base = wid * _PER_W

    pltpu.sync_copy(sf_hbm.at[pl.ds(base, _PER_W)], idx_v)

    def add_offsets(i, carry):
      lanes = base + i * _L + lax.iota(jnp.int32, _L)
      f = lax.rem(lanes, NUM_FIELDS)
      sl = pl.ds(i * _L, _L)
      idx_v[sl] = idx_v[sl] + f * VOCAB
      return carry

    lax.fori_loop(0, _PER_W // _L, add_offsets, 0)

    copies = []
    for j in range(_NCHUNK):
      sl = pl.ds(j * _CHUNK, _CHUNK)
      copies.append(
          pltpu.async_copy(tables_hbm.at[idx_v.at[sl]], rows_v.at[sl], sem)
      )
    for c in copies:
      c.wait()

    pltpu.sync_copy(rows_v, out_hbm.at[pl.ds(base, _PER_W)])

  return body


_gather_kernel = _make_kernel()


@jax.jit
def kernel(sparse_features, tables):
  tables_flat = tables.reshape(NUM_FIELDS * VOCAB, EMB)
  sf_flat = sparse_features.reshape(_TOTAL)
  out_flat = _gather_kernel(tables_flat, sf_flat)
  return out_flat.reshape(BATCH, NUM_FIELDS * EMB)
